# baseline (device time: 27101 ns/iter reference)
import jax
import jax.numpy as jnp
from jax import lax
from jax.experimental import pallas as pl
from jax.experimental.pallas import tpu as pltpu

N_LAYERS = 3
N_CHUNKS = 2


def _dot(a, b):
    return lax.dot_general(
        a, b, (((1,), (0,)), ((), ())), preferred_element_type=jnp.float32
    )


def kernel(x, Win0, Wout0, Win1, Wout1, Win2, Wout2):
    b, d_local = x.shape
    _, h_local = Win0.shape
    chunk = h_local // N_CHUNKS

    def body(x_ref, win0_ref, wout0_ref, win1_ref, wout1_ref, win2_ref,
             wout2_ref, out_ref,
             ysend_ref, yrecv_ref, xsend_ref, xrecv_ref,
             send_sems, recv_sems):
        my_x = lax.axis_index("x")
        my_y = lax.axis_index("y")
        y_nbr = (my_x, 1 - my_y)
        x_nbr = (1 - my_x, my_y)

        barrier_sem = pltpu.get_barrier_semaphore()
        for nbr in (y_nbr, x_nbr):
            pl.semaphore_signal(
                barrier_sem, inc=1,
                device_id=nbr, device_id_type=pl.DeviceIdType.MESH,
            )
        pl.semaphore_wait(barrier_sem, 2)

        wins = (win0_ref, win1_ref, win2_ref)
        wouts = (wout0_ref, wout1_ref, wout2_ref)

        pending_sends = []
        x_cur = x_ref[...]
        for k in range(N_LAYERS):
            win = wins[k][...]
            wout = wouts[k][...]

            rdmas_y = []
            for c in range(N_CHUNKS):
                cols = slice(c * chunk, (c + 1) * chunk)
                ysend_ref[k, c] = _dot(x_cur, win[:, cols])
                rdma = pltpu.make_async_remote_copy(
                    src_ref=ysend_ref.at[k, c],
                    dst_ref=yrecv_ref.at[k, c],
                    send_sem=send_sems.at[3 * k + c],
                    recv_sem=recv_sems.at[3 * k + c],
                    device_id=y_nbr,
                    device_id_type=pl.DeviceIdType.MESH,
                )
                rdma.start()
                rdmas_y.append(rdma)

            p2 = jnp.zeros((b, d_local), jnp.float32)
            for c in range(N_CHUNKS):
                rows = slice(c * chunk, (c + 1) * chunk)
                rdmas_y[c].wait_recv()
                h_c = jnp.maximum(ysend_ref[k, c] + yrecv_ref[k, c], 0.0)
                p2 = p2 + _dot(h_c, wout[rows, :])
            pending_sends.extend(rdmas_y)

            xsend_ref[k] = p2
            rdma_x = pltpu.make_async_remote_copy(
                src_ref=xsend_ref.at[k],
                dst_ref=xrecv_ref.at[k],
                send_sem=send_sems.at[3 * k + 2],
                recv_sem=recv_sems.at[3 * k + 2],
                device_id=x_nbr,
                device_id_type=pl.DeviceIdType.MESH,
            )
            rdma_x.start()
            rdma_x.wait_recv()
            pending_sends.append(rdma_x)
            x_cur = xsend_ref[k] + xrecv_ref[k]

        out_ref[...] = x_cur

        for rdma in pending_sends:
            rdma.wait_send()

    return pl.pallas_call(
        body,
        out_shape=jax.ShapeDtypeStruct((b, d_local), jnp.float32),
        in_specs=[pl.BlockSpec(memory_space=pltpu.VMEM)] * 7,
        out_specs=pl.BlockSpec(memory_space=pltpu.VMEM),
        scratch_shapes=[
            pltpu.VMEM((N_LAYERS, N_CHUNKS, b, chunk), jnp.float32),
            pltpu.VMEM((N_LAYERS, N_CHUNKS, b, chunk), jnp.float32),
            pltpu.VMEM((N_LAYERS, b, d_local), jnp.float32),
            pltpu.VMEM((N_LAYERS, b, d_local), jnp.float32),
            pltpu.SemaphoreType.DMA((3 * N_LAYERS,)),
            pltpu.SemaphoreType.DMA((3 * N_LAYERS,)),
        ],
        compiler_params=pltpu.CompilerParams(collective_id=0),
    )(x, Win0, Wout0, Win1, Wout1, Win2, Wout2)
